# Initial kernel scaffold; baseline (speedup 1.0000x reference)
import jax
import jax.numpy as jnp
from jax.experimental import pallas as pl


def kernel(vertices_feature, edge_index, edge_attr, W1, We1, att_src1, att_dst1, att_edge1, b1, W2, We2, att_src2, att_dst2, att_edge2, b2):
    raise NotImplementedError("write your pallas kernel here")


# speedup vs baseline: 11.3496x; 11.3496x over previous
import functools

import jax
import jax.numpy as jnp
from jax import lax
from jax.experimental import pallas as pl
from jax.experimental.pallas import tpu as pltpu
from jax.experimental.pallas import tpu_sc as plsc

N = 10000
E = 320000
D = 128
DH = 64
D_EDGE = 16

NC = 2
NS = 16
LANES = 16
NW = NC * NS
EC = E // NW
NG = EC // LANES
NP = 10240
SPAN = NP // NS
NB = 5

_SC_MESH = plsc.VectorSubcoreMesh(
    core_axis_name="c", subcore_axis_name="s", num_cores=NC, num_subcores=NS
)


def _sc_gat_body(src_hbm, dst_hbm, ae_hbm, hs_hbm, hd_hbm, hlo_hbm, hhi_hbm,
                 outlo_hbm, outhi_hbm, den_hbm,
                 hs_v, hd_v, src_v, dst_v, ae_v, ex_v, dpart_v,
                 zbuf_v, rows_v, den_sh, osh, *sems):
    sems_g = sems[:NB]
    sems_s = sems[NB:]
    c = lax.axis_index("c")
    s = lax.axis_index("s")
    chunk = c * NS + s
    ebase = chunk * EC
    rbase = s * SPAN
    drows = NP // LANES
    drspan = drows // NS
    last = NS - 1
    zeros16 = jnp.zeros((LANES,), jnp.float32)
    iota16 = lax.iota(jnp.int32, LANES)

    pltpu.sync_copy(hs_hbm, hs_v)
    pltpu.sync_copy(hd_hbm, hd_v)

    def _z1(i, carry):
        dpart_v[i, pl.ds(0, LANES)] = zeros16
        return carry
    lax.fori_loop(0, drows, _z1, 0)

    def _z2(i, carry):
        for cc in range(DH // LANES):
            zbuf_v[i, pl.ds(cc * LANES, LANES)] = zeros16
        return carry
    lax.fori_loop(0, 128, _z2, 0)

    def _zero_own_span():
        for zz in range(SPAN // 128):
            pltpu.sync_copy(zbuf_v, osh.at[pl.ds(rbase + zz * 128, 128)])

    _zero_own_span()
    pltpu.sync_copy(zbuf_v.at[pl.ds(0, drspan), pl.ds(0, LANES)],
                    den_sh.at[pl.ds(s * drspan, drspan)])

    pltpu.sync_copy(src_hbm.at[pl.ds(ebase, EC)], src_v)
    pltpu.sync_copy(dst_hbm.at[pl.ds(ebase, EC)], dst_v)
    pltpu.sync_copy(ae_hbm.at[pl.ds(ebase, EC)], ae_v)

    def _pa(g, carry):
        sl = pl.ds(g * LANES, LANES)
        sv = src_v[sl]
        dv = dst_v[sl]
        al = plsc.load_gather(hs_v, [sv]) + plsc.load_gather(hd_v, [dv]) \
            + ae_v[sl]
        al = jnp.maximum(al, al * 0.2)
        exv = jnp.exp(al)
        ex_v[sl] = exv
        plsc.addupdate_scatter(
            dpart_v,
            [lax.shift_right_logical(dv, 4), lax.bitwise_and(dv, 15)], exv)
        return carry
    lax.fori_loop(0, NG, _pa, 0)

    plsc.subcore_barrier()

    def _pub(k, carry):
        pltpu.sync_copy(dpart_v.at[pl.ds(k * LANES, LANES)],
                        den_sh.at[iota16 + k * LANES], add=True)
        return carry
    lax.fori_loop(0, drows // LANES, _pub, 0)
    plsc.subcore_barrier()

    pltpu.sync_copy(den_sh.at[pl.ds(s * drspan, drspan)],
                    den_hbm.at[c, pl.ds(s * drspan, drspan)])

    def _phase_b(h_half):
        def _issue_gather(g, b):
            sv = src_v[pl.ds(g * LANES, LANES)]
            pltpu.async_copy(h_half.at[sv], rows_v.at[b], sems_g[b])

        for b in range(NB):
            _issue_gather(b, b)

        def _outer(it, carry):
            G = it * NB
            for b in range(NB):
                g = G + b
                pltpu.make_async_copy(
                    h_half.at[pl.ds(0, LANES)], rows_v.at[b],
                    sems_g[b]).wait()
                dv = dst_v[pl.ds(g * LANES, LANES)]

                def _row(i, carry2):
                    bc = plsc.load_gather(
                        ex_v, [jnp.full((LANES,), g * LANES + i, jnp.int32)])
                    for cc in range(DH // LANES):
                        csl = pl.ds(cc * LANES, LANES)
                        rows_v[b, i, csl] = rows_v[b, i, csl] * bc
                    return carry2
                lax.fori_loop(0, LANES, _row, 0)
                pltpu.async_copy(rows_v.at[b], osh.at[dv], sems_s[b],
                                 add=True)
            for b in range(NB):
                g2 = G + NB + b
                pltpu.make_async_copy(
                    rows_v.at[b], osh.at[pl.ds(0, LANES)], sems_s[b]).wait()

                @pl.when(g2 < NG)
                def _():
                    _issue_gather(g2, b)
            return carry
        lax.fori_loop(0, NG // NB, _outer, 0)

    def _copy_out(out_half):
        @pl.when(s < last)
        def _():
            pltpu.sync_copy(osh.at[pl.ds(rbase, SPAN)],
                            out_half.at[c, pl.ds(rbase, SPAN)])

        @pl.when(s == last)
        def _():
            pltpu.sync_copy(osh.at[pl.ds(last * SPAN, N - last * SPAN)],
                            out_half.at[c, pl.ds(last * SPAN,
                                                 N - last * SPAN)])

    _phase_b(hlo_hbm)
    plsc.subcore_barrier()
    _copy_out(outlo_hbm)
    _zero_own_span()
    plsc.subcore_barrier()
    _phase_b(hhi_hbm)
    plsc.subcore_barrier()
    _copy_out(outhi_hbm)


_sc_gat = functools.partial(
    pl.kernel,
    out_type=(jax.ShapeDtypeStruct((NC, N, DH), jnp.float32),
              jax.ShapeDtypeStruct((NC, N, DH), jnp.float32),
              jax.ShapeDtypeStruct((NC, NP // LANES, LANES), jnp.float32)),
    mesh=_SC_MESH,
    compiler_params=pltpu.CompilerParams(needs_layout_passes=False,
                                         use_tc_tiling_on_sc=False),
    scratch_types=[
        pltpu.VMEM((N,), jnp.float32),
        pltpu.VMEM((N,), jnp.float32),
        pltpu.VMEM((EC,), jnp.int32),
        pltpu.VMEM((EC,), jnp.int32),
        pltpu.VMEM((EC,), jnp.float32),
        pltpu.VMEM((EC,), jnp.float32),
        pltpu.VMEM((NP // LANES, LANES), jnp.float32),
        pltpu.VMEM((128, DH), jnp.float32),
        pltpu.VMEM((NB, LANES, DH), jnp.float32),
        pltpu.VMEM_SHARED((NP // LANES, LANES), jnp.float32),
        pltpu.VMEM_SHARED((NP, DH), jnp.float32),
    ] + [pltpu.SemaphoreType.DMA] * (2 * NB),
)(_sc_gat_body)


_BN = 400
_BE = 1280


def _node1_body(x_ref, w_ref, asrc_ref, adst_ref,
                hlo_ref, hhi_ref, hs_ref, hd_ref):
    h = jnp.dot(x_ref[...], w_ref[...], preferred_element_type=jnp.float32)
    hlo_ref[...] = h[:, :DH]
    hhi_ref[...] = h[:, DH:]
    hs_ref[...] = jnp.dot(h, asrc_ref[...], preferred_element_type=jnp.float32)
    hd_ref[...] = jnp.dot(h, adst_ref[...], preferred_element_type=jnp.float32)


def _node2_body(plo0_ref, plo1_ref, phi0_ref, phi1_ref, d0_ref, d1_ref,
                b_ref, w_ref, asrc_ref, adst_ref,
                hlo_ref, hhi_ref, hs_ref, hd_ref):
    den = d0_ref[...] + d1_ref[...] + 1e-16
    num = jnp.concatenate([plo0_ref[0] + plo1_ref[0],
                           phi0_ref[0] + phi1_ref[0]], axis=1)
    xin = num / den + b_ref[...]
    h = jnp.dot(xin, w_ref[...], preferred_element_type=jnp.float32)
    hlo_ref[...] = h[:, :DH]
    hhi_ref[...] = h[:, DH:]
    hs_ref[...] = jnp.dot(h, asrc_ref[...], preferred_element_type=jnp.float32)
    hd_ref[...] = jnp.dot(h, adst_ref[...], preferred_element_type=jnp.float32)


def _comb_body(plo0_ref, plo1_ref, phi0_ref, phi1_ref, d0_ref, d1_ref,
               b_ref, o_ref):
    den = d0_ref[...] + d1_ref[...] + 1e-16
    num = jnp.concatenate([plo0_ref[0] + plo1_ref[0],
                           phi0_ref[0] + phi1_ref[0]], axis=1)
    o_ref[...] = num / den + b_ref[...]


def _edge_body(ea_ref, we1_ref, we2_ref, a1_ref, a2_ref,
               ea2_ref, ae1_ref, ae2_ref):
    blk = ea_ref[...]
    we12 = jnp.dot(we1_ref[...], we2_ref[...],
                   preferred_element_type=jnp.float32)
    v1 = jnp.dot(we1_ref[...], a1_ref[...],
                 preferred_element_type=jnp.float32)
    v2 = jnp.dot(we12, a2_ref[...],
                 preferred_element_type=jnp.float32)
    ea2_ref[...] = jnp.dot(blk, we12, preferred_element_type=jnp.float32)
    ae1_ref[...] = jnp.dot(blk, v1, preferred_element_type=jnp.float32)
    ae2_ref[...] = jnp.dot(blk, v2, preferred_element_type=jnp.float32)


def _full(shape):
    return pl.BlockSpec(shape, lambda i: tuple(0 for _ in shape))


def _half_specs():
    return [pl.BlockSpec((1, _BN, DH), lambda i: (0, i, 0)),
            pl.BlockSpec((1, _BN, DH), lambda i: (1, i, 0))]


_NODE_OUT_SPECS = [
    pl.BlockSpec((_BN, DH), lambda i: (i, 0)),
    pl.BlockSpec((_BN, DH), lambda i: (i, 0)),
    pl.BlockSpec((_BN, 1), lambda i: (i, 0)),
    pl.BlockSpec((_BN, 1), lambda i: (i, 0)),
]

_NODE_OUT_SHAPE = [
    jax.ShapeDtypeStruct((N, DH), jnp.float32),
    jax.ShapeDtypeStruct((N, DH), jnp.float32),
    jax.ShapeDtypeStruct((N, 1), jnp.float32),
    jax.ShapeDtypeStruct((N, 1), jnp.float32),
]


def _node1_call(x, W, asrc, adst):
    return pl.pallas_call(
        _node1_body,
        grid=(N // _BN,),
        in_specs=[
            pl.BlockSpec((_BN, D), lambda i: (i, 0)),
            _full((D, D)), _full((D, 1)), _full((D, 1)),
        ],
        out_specs=_NODE_OUT_SPECS,
        out_shape=_NODE_OUT_SHAPE,
    )(x, W, asrc, adst)


def _node2_call(plo, phi, d0, d1, bias, W, asrc, adst):
    return pl.pallas_call(
        _node2_body,
        grid=(N // _BN,),
        in_specs=_half_specs() + _half_specs() + [
            pl.BlockSpec((_BN, 1), lambda i: (i, 0)),
            pl.BlockSpec((_BN, 1), lambda i: (i, 0)),
            _full((1, D)), _full((D, D)), _full((D, 1)), _full((D, 1)),
        ],
        out_specs=_NODE_OUT_SPECS,
        out_shape=_NODE_OUT_SHAPE,
    )(plo, plo, phi, phi, d0, d1, bias, W, asrc, adst)


def _comb_call(plo, phi, d0, d1, bias):
    return pl.pallas_call(
        _comb_body,
        grid=(N // _BN,),
        in_specs=_half_specs() + _half_specs() + [
            pl.BlockSpec((_BN, 1), lambda i: (i, 0)),
            pl.BlockSpec((_BN, 1), lambda i: (i, 0)),
            _full((1, D)),
        ],
        out_specs=pl.BlockSpec((_BN, D), lambda i: (i, 0)),
        out_shape=jax.ShapeDtypeStruct((N, D), jnp.float32),
    )(plo, plo, phi, phi, d0, d1, bias)


def _edge_call(edge_attr, We1, We2, a1, a2):
    return pl.pallas_call(
        _edge_body,
        grid=(E // _BE,),
        in_specs=[
            pl.BlockSpec((_BE, D_EDGE), lambda i: (i, 0)),
            _full((D_EDGE, D)), _full((D, D)), _full((D, 1)), _full((D, 1)),
        ],
        out_specs=[
            pl.BlockSpec((_BE, D), lambda i: (i, 0)),
            pl.BlockSpec((_BE, 1), lambda i: (i, 0)),
            pl.BlockSpec((_BE, 1), lambda i: (i, 0)),
        ],
        out_shape=[
            jax.ShapeDtypeStruct((E, D), jnp.float32),
            jax.ShapeDtypeStruct((E, 1), jnp.float32),
            jax.ShapeDtypeStruct((E, 1), jnp.float32),
        ],
    )(edge_attr, We1, We2, a1, a2)


def kernel(vertices_feature, edge_index, edge_attr,
           W1, We1, att_src1, att_dst1, att_edge1, b1,
           W2, We2, att_src2, att_dst2, att_edge2, b2):
    src = edge_index[0]
    dst = edge_index[1]

    h1lo, h1hi, hs1, hd1 = _node1_call(vertices_feature, W1,
                                       att_src1[:, None], att_dst1[:, None])
    ea2, ae1, ae2 = _edge_call(edge_attr, We1, We2,
                               att_edge1[:, None], att_edge2[:, None])

    p1lo, p1hi, den1 = _sc_gat(src, dst, ae1.reshape(E),
                               hs1.reshape(N), hd1.reshape(N), h1lo, h1hi)
    den1r = den1.reshape(NC, NP)
    h2lo, h2hi, hs2, hd2 = _node2_call(
        p1lo, p1hi, den1r[0, :N, None], den1r[1, :N, None],
        b1[None, :], W2, att_src2[:, None], att_dst2[:, None])

    p2lo, p2hi, den2 = _sc_gat(src, dst, ae2.reshape(E),
                               hs2.reshape(N), hd2.reshape(N), h2lo, h2hi)
    den2r = den2.reshape(NC, NP)
    out = _comb_call(p2lo, p2hi, den2r[0, :N, None], den2r[1, :N, None],
                     b2[None, :])
    return (out, ea2)
